# baseline (speedup 1.0000x reference)
import jax
import jax.numpy as jnp
from jax import lax
from jax.experimental import pallas as pl
from jax.experimental.pallas import tpu as pltpu


def _tc_body(coords_smem, agent_vmem, in_hbm, out_hbm, slab_vmem,
             bulk_sems, slab_sem):
    n_s, ch, h, w = in_hbm.shape
    c0 = coords_smem[0]
    c1 = coords_smem[1]
    sid = coords_smem[2]

    bulk = [
        pltpu.make_async_copy(in_hbm.at[s], out_hbm.at[s], bulk_sems.at[s])
        for s in range(n_s)
    ]
    for c in bulk:
        c.start()

    gather = pltpu.make_async_copy(in_hbm.at[sid, :, c0, :], slab_vmem,
                                   slab_sem)
    gather.start()
    gather.wait()

    col = lax.broadcasted_iota(jnp.int32, (ch, w), 1)
    slab = slab_vmem[...]
    slab_vmem[...] = jnp.where(col == c1,
                               jnp.maximum(slab, agent_vmem[...]), slab)

    pltpu.make_async_copy(in_hbm.at[sid], out_hbm.at[sid],
                          bulk_sems.at[sid]).wait()
    scatter = pltpu.make_async_copy(slab_vmem, out_hbm.at[sid, :, c0, :],
                                    slab_sem)
    scatter.start()
    scatter.wait()

    for s in range(n_s):
        @pl.when(jnp.int32(s) != sid)
        def _():
            pltpu.make_async_copy(in_hbm.at[s], out_hbm.at[s],
                                  bulk_sems.at[s]).wait()


def kernel(input_grid, input_state_of_agent, coordinates_at_last_frame, scene_id):
    s, ch, h, w = input_grid.shape
    coords = jnp.stack([
        coordinates_at_last_frame[0].astype(jnp.int32),
        coordinates_at_last_frame[1].astype(jnp.int32),
        jnp.asarray(scene_id, jnp.int32),
        jnp.int32(0),
    ])
    agent = input_state_of_agent.reshape(ch, 1).astype(jnp.float32)

    return pl.pallas_call(
        _tc_body,
        out_shape=jax.ShapeDtypeStruct((s, ch, h, w), jnp.float32),
        in_specs=[
            pl.BlockSpec(memory_space=pltpu.SMEM),
            pl.BlockSpec(memory_space=pltpu.VMEM),
            pl.BlockSpec(memory_space=pltpu.MemorySpace.HBM),
        ],
        out_specs=pl.BlockSpec(memory_space=pltpu.MemorySpace.HBM),
        scratch_shapes=[
            pltpu.VMEM((ch, w), jnp.float32),
            pltpu.SemaphoreType.DMA((s,)),
            pltpu.SemaphoreType.DMA,
        ],
    )(coords, agent, input_grid)


# speedup vs baseline: 41.9873x; 41.9873x over previous
import jax
import jax.numpy as jnp
from jax import lax
from jax.experimental import pallas as pl
from jax.experimental.pallas import tpu as pltpu

_CB = 8


def _tc_body(coords_smem, agent_vmem, x_ref, o_ref):
    i = pl.program_id(0)
    n_cb = pl.num_programs(1)
    j = pl.program_id(1)
    c0 = coords_smem[0]
    c1 = coords_smem[1]
    sid = coords_smem[2]

    o_ref[...] = x_ref[...]

    @pl.when(i == sid)
    def _():
        blk = o_ref[...]
        row = lax.broadcasted_iota(jnp.int32, blk.shape, 2)
        col = lax.broadcasted_iota(jnp.int32, blk.shape, 3)
        agent = agent_vmem[pl.ds(j * _CB, _CB), :]
        agent = agent.reshape(1, _CB, 1, 1)
        o_ref[...] = jnp.where((row == c0) & (col == c1),
                               jnp.maximum(blk, agent), blk)


def kernel(input_grid, input_state_of_agent, coordinates_at_last_frame, scene_id):
    s, ch, h, w = input_grid.shape
    coords = jnp.stack([
        coordinates_at_last_frame[0].astype(jnp.int32),
        coordinates_at_last_frame[1].astype(jnp.int32),
        jnp.asarray(scene_id, jnp.int32),
        jnp.int32(0),
    ])
    agent = input_state_of_agent.reshape(ch, 1).astype(jnp.float32)

    blk = (1, _CB, h, w)
    return pl.pallas_call(
        _tc_body,
        grid=(s, ch // _CB),
        out_shape=jax.ShapeDtypeStruct((s, ch, h, w), jnp.float32),
        in_specs=[
            pl.BlockSpec(memory_space=pltpu.SMEM),
            pl.BlockSpec(memory_space=pltpu.VMEM),
            pl.BlockSpec(blk, lambda i, j: (i, j, 0, 0)),
        ],
        out_specs=pl.BlockSpec(blk, lambda i, j: (i, j, 0, 0)),
    )(coords, agent, input_grid)
